# baseline (speedup 1.0000x reference)
import functools

import jax
import jax.numpy as jnp
from jax import lax
from jax.experimental import pallas as pl
from jax.experimental.pallas import tpu as pltpu
from jax.experimental.pallas import tpu_sc as plsc

B = 16384
D = 32
NC, NS, L = 2, 16, 16
NW = NC * NS
NU = 1000000
NI = 100000
BLK = 128
UBF = NU // BLK
IBF = NI // BLK
UBLKS = UBF + 1
IBLKS = IBF + 1
UPW = -(-UBLKS // NW)
IPW = -(-IBLKS // NW)
CAP = 1024
NIDV = B // L
NWROWS = 2 * D + 1


def _body(uid_hbm, iid_hbm, ut_hbm, it_hbm, utail_hbm, itail_hbm, w_hbm,
          up_hbm, ip_hbm,
          ids_u, ids_i, w_v, mem_id, mem_pos, mem_val, sc_pos, sc_val,
          blkbuf, tail_v, sem_blk, sem_sc):
    wid = lax.axis_index("s") * NC + lax.axis_index("c")
    pltpu.sync_copy(uid_hbm, ids_u)
    pltpu.sync_copy(iid_hbm, ids_i)
    pltpu.sync_copy(w_hbm, w_v)

    lanes = lax.iota(jnp.int32, L)
    bias_row = w_v[pl.ds(2 * D * L, L)]
    zero_row = jnp.zeros((L,), jnp.float32)

    def memset_members():
        def st(v, c):
            mem_id[pl.ds(v * L, L)] = jnp.full((L,), 0x7FFFFFF, jnp.int32)
            mem_pos[pl.ds(v * L, L)] = jnp.full((L,), B, jnp.int32)
            return c
        lax.fori_loop(0, CAP // L, st, 0)

    def build_members(ids_ref, lo_blk, n_blk):
        def step(v, off):
            idv = ids_ref[pl.ds(v * L, L)]
            ub = lax.shift_right_logical(idv, 7)
            m = (ub >= lo_blk) & (ub < lo_blk + n_blk)
            posv = v * L + lanes
            plsc.store_compressed(mem_id.at[pl.ds(off, L)], idv, mask=m)
            plsc.store_compressed(mem_pos.at[pl.ds(off, L)], posv, mask=m)
            cnt = plsc.all_reduce_population_count(m)
            return off + cnt[0]
        return lax.fori_loop(0, NIDV, step, 0)

    def compute_members(nv, blk, acc0, wbase, gather):
        def mstep(v, carry):
            idv = mem_id[pl.ds(v * L, L)]
            m = lax.shift_right_logical(idv, 7) == blk

            @pl.when(jnp.any(m))
            def _():
                col = jnp.bitwise_and(idv, BLK - 1)
                acc = acc0
                for d in range(D):
                    acc = acc + gather(d, col) * w_v[pl.ds((wbase + d) * L, L)]
                old = mem_val[pl.ds(v * L, L)]
                mem_val[pl.ds(v * L, L)] = jnp.where(m, acc, old)
            return carry
        lax.fori_loop(0, nv, mstep, 0)

    def scan_table(tab_hbm, n_blocks_global, per_worker, ids_ref, acc0, wbase,
                   tail_blk):
        lo_blk = wid * per_worker
        n_total = jnp.clip(n_blocks_global - lo_blk, 0, per_worker)
        cnt = build_members(ids_ref, lo_blk, n_total)
        nv = (cnt + L - 1) // L
        has_tail = lo_blk + n_total == n_blocks_global
        n_full = jnp.where(has_tail, n_total - 1, n_total)

        def issue(blk, parity):
            pltpu.make_async_copy(
                tab_hbm.at[:, pl.ds(pl.multiple_of(blk * BLK, BLK), BLK)],
                blkbuf.at[parity], sem_blk).start()

        def drain(parity):
            pltpu.make_async_copy(
                tab_hbm.at[:, pl.ds(0, BLK)], blkbuf.at[parity],
                sem_blk).wait()

        @pl.when(n_full > 0)
        def _():
            issue(lo_blk, 0)

        def gstep(g, carry):
            @pl.when(g + 1 < n_full)
            def _():
                issue(lo_blk + g + 1, (g + 1) % 2)
            drain(g % 2)
            kvec = jnp.full((L,), g % 2, jnp.int32)

            def gather(d, col):
                return plsc.load_gather(
                    blkbuf, [kvec, jnp.full((L,), d, jnp.int32), col])
            compute_members(nv, lo_blk + g, acc0, wbase, gather)
            return carry
        lax.fori_loop(0, n_full, gstep, 0)

        @pl.when(has_tail & (n_total > 0))
        def _():
            def gather(d, col):
                return plsc.load_gather(
                    tail_v, [jnp.full((L,), d, jnp.int32), col])
            compute_members(nv, tail_blk, acc0, wbase, gather)

    def scatter_members(part_hbm):
        for j in range(CAP // BLK):
            for q in range(BLK // L):
                sc_pos[j, pl.ds(q * L, L)] = mem_pos[pl.ds(j * BLK + q * L, L)]
                sc_val[j, pl.ds(q * L, L)] = mem_val[pl.ds(j * BLK + q * L, L)]
        cps = [pltpu.async_copy(sc_val.at[j], part_hbm.at[sc_pos.at[j]],
                                sem_sc)
               for j in range(CAP // BLK)]
        for cp in cps:
            cp.wait()

    pltpu.sync_copy(utail_hbm, tail_v)
    memset_members()
    scan_table(ut_hbm, UBLKS, UPW, ids_u, bias_row, 0, UBF)
    scatter_members(up_hbm)

    pltpu.sync_copy(itail_hbm, tail_v)
    memset_members()
    scan_table(it_hbm, IBLKS, IPW, ids_i, zero_row, D, IBF)
    scatter_members(ip_hbm)


_mesh = plsc.VectorSubcoreMesh(core_axis_name="c", subcore_axis_name="s")

_ncf = functools.partial(
    pl.kernel, mesh=_mesh,
    compiler_params=pltpu.CompilerParams(
        needs_layout_passes=False, use_tc_tiling_on_sc=True),
    out_type=(jax.ShapeDtypeStruct((B + BLK,), jnp.float32),
              jax.ShapeDtypeStruct((B + BLK,), jnp.float32)),
    scratch_types=[
        pltpu.VMEM((B,), jnp.int32),
        pltpu.VMEM((B,), jnp.int32),
        pltpu.VMEM((NWROWS * L,), jnp.float32),
        pltpu.VMEM((CAP,), jnp.int32),
        pltpu.VMEM((CAP,), jnp.int32),
        pltpu.VMEM((CAP,), jnp.float32),
        pltpu.VMEM((CAP // BLK, BLK), jnp.int32),
        pltpu.VMEM((CAP // BLK, BLK), jnp.float32),
        pltpu.VMEM((2, D, BLK), jnp.float32),
        pltpu.VMEM((D, BLK), jnp.float32),
        pltpu.SemaphoreType.DMA,
        pltpu.SemaphoreType.DMA,
    ],
)(_body)


def kernel(user_ids, item_ids, user_table, item_table, fc_w, fc_b):
    uid = user_ids.astype(jnp.int32)
    iid = item_ids.astype(jnp.int32)
    ut_t = user_table.T
    it_t = item_table.T
    utail = jnp.pad(user_table[UBF * BLK:].T, ((0, 0), (0, BLK - (NU - UBF * BLK))))
    itail = jnp.pad(item_table[IBF * BLK:].T, ((0, 0), (0, BLK - (NI - IBF * BLK))))
    w_all = jnp.repeat(
        jnp.concatenate([fc_w.reshape(-1), fc_b.reshape(-1)]).astype(jnp.float32),
        L,
    )
    upart, ipart = _ncf(uid, iid, ut_t, it_t, utail, itail, w_all)
    return (upart[:B] + ipart[:B]).reshape(B, 1)
